# baseline (speedup 1.0000x reference)
import functools

import jax
import jax.numpy as jnp
from jax import lax
from jax.experimental import pallas as pl
from jax.experimental.pallas import tpu as pltpu
from jax.experimental.pallas import tpu_sc as plsc

_EMB_NUM = 8192
_EMB_DIM = 32
_BETA = 0.25
_N_TOK = 4096
_TILE_T = 512
_TILE_K = 4096
_N_T = _N_TOK // _TILE_T
_N_K = _EMB_NUM // _TILE_K


def _assign_body(z_ref, emb_ref, idx_ref, loss_ref, perp_ref, counts_scr,
                 loss_scr, esq_scr):
    step = pl.program_id(0)

    @pl.when(step == 0)
    def _init():
        counts_scr[...] = jnp.zeros_like(counts_scr)
        loss_scr[0] = 0.0
        e0 = emb_ref[...]
        esq_scr[:, 0] = jnp.ones((_EMB_NUM,), jnp.float32)
        esq_scr[:, 1] = jnp.sum(e0 * e0, axis=1)

    z = z_ref[...]
    zsq = jnp.sum(z * z, axis=1, keepdims=True)
    zs1 = jnp.concatenate(
        [zsq, jnp.ones((_TILE_T, 1), jnp.float32)], axis=1)
    col_iota = lax.broadcasted_iota(jnp.int32, (_TILE_T, _TILE_K), 1)
    best_d = None
    best_i = None
    for c in range(_N_K):
        e = emb_ref[pl.ds(c * _TILE_K, _TILE_K), :]
        m = lax.dot_general(z, e, (((1,), (1,)), ((), ())),
                            preferred_element_type=jnp.float32)
        s = lax.dot_general(zs1, esq_scr[pl.ds(c * _TILE_K, _TILE_K), :],
                            (((1,), (1,)), ((), ())),
                            preferred_element_type=jnp.float32)
        d = s - 2.0 * m
        cmin = jnp.min(d, axis=1)
        first = jnp.min(
            jnp.where(d == cmin[:, None], col_iota, _EMB_NUM), axis=1)
        if c == 0:
            best_d, best_i = cmin, first
        else:
            upd = cmin < best_d
            best_d = jnp.where(upd, cmin, best_d)
            best_i = jnp.where(upd, first + c * _TILE_K, best_i)
    idx_ref[0, 0, :] = best_i

    loss_scr[0] += jnp.sum(best_d)
    hi = lax.shift_right_logical(best_i, 7)
    lo = jnp.bitwise_and(best_i, 127)
    oh_hi = (hi[:, None] == lax.broadcasted_iota(
        jnp.int32, (_TILE_T, _EMB_NUM // 128), 1)).astype(jnp.float32)
    oh_lo = (lo[:, None] == lax.broadcasted_iota(
        jnp.int32, (_TILE_T, 128), 1)).astype(jnp.float32)
    counts_scr[...] += lax.dot_general(
        oh_hi, oh_lo, (((0,), (0,)), ((), ())),
        preferred_element_type=jnp.float32)

    @pl.when(step == _N_T - 1)
    def _finish():
        e_mean = counts_scr[...] * (1.0 / _N_TOK)
        ent = jnp.sum(e_mean * jnp.log(e_mean + 1e-10))
        perp_ref[0, 0] = jnp.exp(-ent)
        loss_ref[0, 0] = loss_scr[0] * ((1.0 + _BETA) / (_N_TOK * _EMB_DIM))


_assign = pl.pallas_call(
    _assign_body,
    grid=(_N_T,),
    in_specs=[
        pl.BlockSpec((_TILE_T, _EMB_DIM), lambda i: (i, 0)),
        pl.BlockSpec((_EMB_NUM, _EMB_DIM), lambda i: (0, 0)),
    ],
    out_specs=[
        pl.BlockSpec((1, 1, _TILE_T), lambda i: (i, 0, 0)),
        pl.BlockSpec(memory_space=pltpu.SMEM),
        pl.BlockSpec(memory_space=pltpu.SMEM),
    ],
    out_shape=[
        jax.ShapeDtypeStruct((_N_T, 1, _TILE_T), jnp.int32),
        jax.ShapeDtypeStruct((1, 1), jnp.float32),
        jax.ShapeDtypeStruct((1, 1), jnp.float32),
    ],
    scratch_shapes=[
        pltpu.VMEM((_EMB_NUM // 128, 128), jnp.float32),
        pltpu.SMEM((1,), jnp.float32),
        pltpu.VMEM((_EMB_NUM, 2), jnp.float32),
    ],
)

_NC, _NS = 2, 16
_NW = _NC * _NS
_BPW = _N_TOK // _NW


@functools.cache
def _make_gather():
    mesh = plsc.VectorSubcoreMesh(core_axis_name="c", subcore_axis_name="s",
                                  num_cores=_NC, num_subcores=_NS)

    @functools.partial(
        pl.kernel,
        mesh=mesh,
        out_type=jax.ShapeDtypeStruct((_N_TOK, _EMB_DIM), jnp.float32),
        scratch_types=[
            pltpu.VMEM((_BPW,), jnp.int32),
            pltpu.VMEM((_BPW, _EMB_DIM), jnp.float32),
            pltpu.SemaphoreType.DMA,
        ],
        compiler_params=pltpu.CompilerParams(use_tc_tiling_on_sc=False),
    )
    def _gather_rows(table_hbm, idx_hbm, out_hbm, idx_v, rows_v, sem):
        wid = lax.axis_index("s") * _NC + lax.axis_index("c")
        base = wid * _BPW
        pltpu.sync_copy(idx_hbm.at[pl.ds(base, _BPW)], idx_v)
        pltpu.async_copy(table_hbm.at[idx_v], rows_v, sem).wait()
        pltpu.sync_copy(rows_v, out_hbm.at[pl.ds(base, _BPW)])

    return _gather_rows


def kernel(z, emb_weight):
    z_flat = z.reshape(-1, _EMB_DIM)
    idx3, loss11, perp11 = _assign(z_flat, emb_weight)
    idx = idx3.reshape(_N_TOK)
    z_q = _make_gather()(emb_weight, idx).reshape(z.shape)
    z_q_st = z + lax.stop_gradient(z_q - z)
    return (loss11[0, 0], z_q_st, emb_weight, perp11[0, 0])


# speedup vs baseline: 1.2575x; 1.2575x over previous
import functools

import jax
import jax.numpy as jnp
from jax import lax
from jax.experimental import pallas as pl
from jax.experimental.pallas import tpu as pltpu
from jax.experimental.pallas import tpu_sc as plsc

_EMB_NUM = 8192
_EMB_DIM = 32
_BETA = 0.25
_N_TOK = 4096
_TILE_T = 1024
_TILE_K = 8192
_N_T = _N_TOK // _TILE_T
_N_K = _EMB_NUM // _TILE_K


def _assign_body(z_ref, emb_ref, idx_ref, loss_ref, perp_ref, counts_scr,
                 loss_scr, esq_scr):
    step = pl.program_id(0)

    @pl.when(step == 0)
    def _init():
        counts_scr[...] = jnp.zeros_like(counts_scr)
        loss_scr[0] = 0.0
        e0 = emb_ref[...]
        esq_scr[0, :] = jnp.sum(e0 * e0, axis=1)

    z = z_ref[...]
    zsq = jnp.sum(z * z, axis=1, keepdims=True)
    col_iota = lax.broadcasted_iota(jnp.int32, (_TILE_T, _EMB_NUM), 1)
    e = emb_ref[...]
    esq = esq_scr[0, :]
    m = lax.dot_general(z, e, (((1,), (1,)), ((), ())),
                        preferred_element_type=jnp.float32)
    d = (zsq + esq[None, :]) - 2.0 * m
    best_d = jnp.min(d, axis=1)
    best_i = jnp.min(
        jnp.where(d == best_d[:, None], col_iota, _EMB_NUM), axis=1)
    idx_ref[0, 0, :] = best_i

    loss_scr[0] += jnp.sum(best_d)
    hi = lax.shift_right_logical(best_i, 7)
    lo = jnp.bitwise_and(best_i, 127)
    oh_hi = (hi[:, None] == lax.broadcasted_iota(
        jnp.int32, (_TILE_T, _EMB_NUM // 128), 1)).astype(jnp.float32)
    oh_lo = (lo[:, None] == lax.broadcasted_iota(
        jnp.int32, (_TILE_T, 128), 1)).astype(jnp.float32)
    counts_scr[...] += lax.dot_general(
        oh_hi, oh_lo, (((0,), (0,)), ((), ())),
        preferred_element_type=jnp.float32)

    @pl.when(step == _N_T - 1)
    def _finish():
        e_mean = counts_scr[...] * (1.0 / _N_TOK)
        ent = jnp.sum(e_mean * jnp.log(e_mean + 1e-10))
        perp_ref[0, 0] = jnp.exp(-ent)
        loss_ref[0, 0] = loss_scr[0] * ((1.0 + _BETA) / (_N_TOK * _EMB_DIM))


_assign = pl.pallas_call(
    _assign_body,
    grid=(_N_T,),
    in_specs=[
        pl.BlockSpec((_TILE_T, _EMB_DIM), lambda i: (i, 0)),
        pl.BlockSpec((_EMB_NUM, _EMB_DIM), lambda i: (0, 0)),
    ],
    out_specs=[
        pl.BlockSpec((1, 1, _TILE_T), lambda i: (i, 0, 0)),
        pl.BlockSpec(memory_space=pltpu.SMEM),
        pl.BlockSpec(memory_space=pltpu.SMEM),
    ],
    out_shape=[
        jax.ShapeDtypeStruct((_N_T, 1, _TILE_T), jnp.int32),
        jax.ShapeDtypeStruct((1, 1), jnp.float32),
        jax.ShapeDtypeStruct((1, 1), jnp.float32),
    ],
    scratch_shapes=[
        pltpu.VMEM((_EMB_NUM // 128, 128), jnp.float32),
        pltpu.SMEM((1,), jnp.float32),
        pltpu.VMEM((1, _EMB_NUM), jnp.float32),
    ],
)

_NC, _NS = 2, 16
_NW = _NC * _NS
_BPW = _N_TOK // _NW


@functools.cache
def _make_gather():
    mesh = plsc.VectorSubcoreMesh(core_axis_name="c", subcore_axis_name="s",
                                  num_cores=_NC, num_subcores=_NS)

    @functools.partial(
        pl.kernel,
        mesh=mesh,
        out_type=jax.ShapeDtypeStruct((_N_TOK, _EMB_DIM), jnp.float32),
        scratch_types=[
            pltpu.VMEM((_BPW,), jnp.int32),
            pltpu.VMEM((_BPW, _EMB_DIM), jnp.float32),
            pltpu.SemaphoreType.DMA,
        ],
        compiler_params=pltpu.CompilerParams(use_tc_tiling_on_sc=False),
    )
    def _gather_rows(table_hbm, idx_hbm, out_hbm, idx_v, rows_v, sem):
        wid = lax.axis_index("s") * _NC + lax.axis_index("c")
        base = wid * _BPW
        pltpu.sync_copy(idx_hbm.at[pl.ds(base, _BPW)], idx_v)
        pltpu.async_copy(table_hbm.at[idx_v], rows_v, sem).wait()
        pltpu.sync_copy(rows_v, out_hbm.at[pl.ds(base, _BPW)])

    return _gather_rows


def kernel(z, emb_weight):
    z_flat = z.reshape(-1, _EMB_DIM)
    idx3, loss11, perp11 = _assign(z_flat, emb_weight)
    idx = idx3.reshape(_N_TOK)
    z_q = _make_gather()(emb_weight, idx).reshape(z.shape)
    z_q_st = z + lax.stop_gradient(z_q - z)
    return (loss11[0, 0], z_q_st, emb_weight, perp11[0, 0])
